# Initial kernel scaffold; baseline (speedup 1.0000x reference)
import jax
import jax.numpy as jnp
from jax.experimental import pallas as pl


def kernel(z, vector_features, edge_index, edge_attr, pos, residue_index, is_ligand, mask_interface, batch_idx):
    raise NotImplementedError("write your pallas kernel here")


# speedup vs baseline: 13.5105x; 13.5105x over previous
import functools

import jax
import jax.numpy as jnp
import numpy as np
from jax import lax
from jax.experimental import pallas as pl
from jax.experimental.pallas import tpu as pltpu
from jax.experimental.pallas import tpu_sc as plsc

_EDGE_DIM = 19
_NUM_GAUSSIANS = _EDGE_DIM - 3
_MASK_NOISE = 0.5
_NUM_GRAPHS = 8

_N = 50000
_E = 1600000

_NR, _NC = 8, 6272
_NPAD2 = _NR * _NC

_NP = 50048
_DUMMY = _NP
_MROWS = 50176
_MC = 24

_SUB = 128
_EPT = 100352
_EP = 16 * _EPT
_NSUB = _EPT // _SUB

_RBF_OFFSET = np.linspace(0.0, 10.0, _NUM_GAUSSIANS)
_RBF_COEFF = -0.5 / float(_RBF_OFFSET[1] - _RBF_OFFSET[0]) ** 2


def _mask_body(in_ref, out_ref):
    x = in_ref[0]
    y = in_ref[1]
    z = in_ref[2]
    b = in_ref[3]
    lig = in_ref[4]
    idxr = in_ref[5]
    shape = x.shape
    masks = [jnp.zeros(shape, jnp.float32), jnp.zeros(shape, jnp.float32)]
    for g in range(_NUM_GRAPHS):
        gm = b == float(g)
        for l in (0, 1):
            chain = gm & (lig == float(l))
            cf = chain.astype(jnp.float32)
            cnt = jnp.sum(cf)
            denom = jnp.maximum(cnt, 1.0)
            cx = jnp.sum(x * cf) / denom
            cy = jnp.sum(y * cf) / denom
            cz = jnp.sum(z * cf) / denom
            dx = x - cx
            dy = y - cy
            dz = z - cz
            dc = jnp.sqrt(dx * dx + dy * dy + dz * dz)
            dc = jnp.where(chain, dc, jnp.inf)
            mn = jnp.min(dc)
            cand = jnp.where(dc == mn, idxr, 3.0e38)
            cidx = jnp.min(cand)
            sel = (idxr == cidx).astype(jnp.float32)
            px = jnp.sum(x * sel)
            py = jnp.sum(y * sel)
            pz = jnp.sum(z * sel)
            ex = x - px
            ey = y - py
            ez = z - pz
            d = jnp.sqrt(ex * ex + ey * ey + ez * ez)
            local = chain & (d < 10.0) & (cnt > 0.0)
            masks[l] = jnp.maximum(masks[l], local.astype(jnp.float32))
    out_ref[0] = masks[0]
    out_ref[1] = masks[1]


_mask_call = pl.pallas_call(
    _mask_body,
    out_shape=jax.ShapeDtypeStruct((2, _NR, _NC), jnp.float32),
)


def _edge_body(rows_hbm, cols_hbm, et0_hbm, et1_hbm, et2_hbm, posv_hbm,
               out_hbm, rows_v, cols_v, cols2_v, e0_v, e1_v, e2_v,
               pr_v, pc_v, feat_v, posv_sh, msg_sh):
    c = lax.axis_index("c")
    s = lax.axis_index("s")

    prows = _NP // 16
    pltpu.sync_copy(posv_hbm.at[pl.ds(s * prows, prows)],
                    posv_sh.at[pl.ds(s * prows, prows)])

    lanes = lax.iota(jnp.int32, 16)
    zeros16 = jnp.zeros((16,), jnp.float32)
    for i8 in range(_SUB // 16):
        for k in range(_MC):
            plsc.store_scatter(
                feat_v, [lanes + i8 * 16, jnp.full((16,), k, jnp.int32)],
                zeros16)
    zrows = _MROWS // 16
    zbase = s * zrows
    for j in range(zrows // _SUB):
        pltpu.sync_copy(feat_v, msg_sh.at[pl.ds(zbase + j * _SUB, _SUB)])
    pltpu.sync_copy(feat_v.at[pl.ds(0, 64)],
                    msg_sh.at[pl.ds(zbase + (zrows // _SUB) * _SUB, 64)])
    plsc.subcore_barrier()

    zi = jnp.zeros((16,), jnp.int32)
    base = c * 3
    kx = zi + base
    ky = zi + base + 1
    kz = zi + base + 2
    kl = zi + 6

    def sub_body(i, carry):
        r = s * _NSUB + i
        pltpu.sync_copy(rows_hbm.at[pl.ds(r * _SUB, _SUB)], rows_v.at[0])
        pltpu.sync_copy(cols_hbm.at[pl.ds(r * _SUB, _SUB)], cols_v.at[0])
        pltpu.sync_copy(et0_hbm.at[pl.ds(r * _SUB, _SUB)], e0_v)
        pltpu.sync_copy(et1_hbm.at[pl.ds(r * _SUB, _SUB)], e1_v)
        pltpu.sync_copy(et2_hbm.at[pl.ds(r * _SUB, _SUB)], e2_v)
        pltpu.sync_copy(posv_sh.at[rows_v.at[0]], pr_v)
        pltpu.sync_copy(posv_sh.at[cols_v.at[0]], pc_v)
        for i8 in range(_SUB // 16):
            o = i8 * 16
            idx = lanes + o
            xr = plsc.load_gather(pr_v, [idx, kx])
            yr = plsc.load_gather(pr_v, [idx, ky])
            zr = plsc.load_gather(pr_v, [idx, kz])
            lr = plsc.load_gather(pr_v, [idx, kl])
            xc = plsc.load_gather(pc_v, [idx, kx])
            yc = plsc.load_gather(pc_v, [idx, ky])
            zc = plsc.load_gather(pc_v, [idx, kz])
            lc = plsc.load_gather(pc_v, [idx, kl])
            dx = xr - xc
            dy = yr - yc
            dz = zr - zc
            d2 = dx * dx + dy * dy + dz * dz + 1e-12
            u = lax.bitcast_convert_type(d2, jnp.int32)
            u = jnp.int32(0x5F3759DF) - (u >> 1)
            yv = lax.bitcast_convert_type(u, jnp.float32)
            for _ in range(3):
                yv = yv * (1.5 - 0.5 * d2 * yv * yv)
            d = d2 * yv
            cv = cols_v[0, pl.ds(o, 16)]
            same = lr == lc
            cols2_v[0, pl.ds(o, 16)] = jnp.where(same, cv, jnp.int32(_DUMMY))
            e0 = e0_v[pl.ds(o, 16)]
            e1 = e1_v[pl.ds(o, 16)]
            e2 = e2_v[pl.ds(o, 16)]
            plsc.store_scatter(feat_v, [idx, zi], e0)
            plsc.store_scatter(feat_v, [idx, zi + 1], e1)
            plsc.store_scatter(feat_v, [idx, zi + 2], e2)
            for k in range(_NUM_GAUSSIANS):
                t = d - jnp.float32(_RBF_OFFSET[k])
                val = jnp.exp(jnp.float32(_RBF_COEFF) * t * t)
                plsc.store_scatter(feat_v, [idx, zi + (3 + k)], val)
        pltpu.sync_copy(feat_v, msg_sh.at[cols2_v.at[0]], add=True)
        return carry

    lax.fori_loop(0, _NSUB, sub_body, 0)
    plsc.subcore_barrier()

    orows = _NP // 16
    pltpu.sync_copy(msg_sh.at[pl.ds(s * orows, orows)],
                    out_hbm.at[c, pl.ds(s * orows, orows)])


_edge_call = pl.kernel(
    _edge_body,
    out_type=jax.ShapeDtypeStruct((2, _NP, _MC), jnp.float32),
    mesh=plsc.VectorSubcoreMesh(core_axis_name="c", subcore_axis_name="s"),
    compiler_params=pltpu.CompilerParams(
        needs_layout_passes=False, use_tc_tiling_on_sc=False),
    scratch_types=[
        pltpu.VMEM((1, _SUB), jnp.int32),
        pltpu.VMEM((1, _SUB), jnp.int32),
        pltpu.VMEM((1, _SUB), jnp.int32),
        pltpu.VMEM((_SUB,), jnp.float32),
        pltpu.VMEM((_SUB,), jnp.float32),
        pltpu.VMEM((_SUB,), jnp.float32),
        pltpu.VMEM((_SUB, 8), jnp.float32),
        pltpu.VMEM((_SUB, 8), jnp.float32),
        pltpu.VMEM((_SUB, _MC), jnp.float32),
        pltpu.VMEM_SHARED((_NP, 8), jnp.float32),
        pltpu.VMEM_SHARED((_MROWS, _MC), jnp.float32),
    ],
)


def _plane(a, pad_val):
    a = a.astype(jnp.float32)
    return jnp.pad(a, (0, _NPAD2 - _N), constant_values=pad_val).reshape(_NR, _NC)


@jax.jit
def kernel(z, vector_features, edge_index, edge_attr, pos, residue_index,
           is_ligand, mask_interface, batch_idx):
    del z, vector_features, residue_index, mask_interface
    nk = jax.random.key(1)
    n1 = jax.random.normal(jax.random.fold_in(nk, 1), (_N, 3), jnp.float32) * _MASK_NOISE
    n2 = jax.random.normal(jax.random.fold_in(nk, 2), (_N, 3), jnp.float32) * _MASK_NOISE

    planes = jnp.stack([
        _plane(pos[:, 0], 0.0),
        _plane(pos[:, 1], 0.0),
        _plane(pos[:, 2], 0.0),
        _plane(batch_idx, -1.0),
        _plane(is_ligand, -1.0),
        _plane(jnp.arange(_N, dtype=jnp.float32), 3.0e38),
    ])
    masks = _mask_call(planes)
    m1 = (masks[0].reshape(-1)[:_N] > 0.0)[:, None]
    m2 = (masks[1].reshape(-1)[:_N] > 0.0)[:, None]
    pos_v1 = jnp.where(m1, n1, pos)
    pos_v2 = jnp.where(m2, n2, pos)

    ligf = is_ligand.astype(jnp.float32)[:, None]
    pad = jnp.zeros((_N, 1), jnp.float32)
    pv = jnp.concatenate([pos_v1, pos_v2, ligf, pad], axis=1)
    posv = jnp.pad(pv, ((0, _NP - _N), (0, 0)))

    row = edge_index[0].astype(jnp.int32)
    col = edge_index[1].astype(jnp.int32)
    rows1d = jnp.pad(row, (0, _EP - _E))
    cols1d = jnp.pad(col, (0, _EP - _E), constant_values=_DUMMY)
    et0 = jnp.pad(edge_attr[:, 0], (0, _EP - _E))
    et1 = jnp.pad(edge_attr[:, 1], (0, _EP - _E))
    et2 = jnp.pad(edge_attr[:, 2], (0, _EP - _E))
    msg = _edge_call(rows1d, cols1d, et0, et1, et2, posv)
    return jnp.concatenate(
        [pos_v1, pos_v2, msg[0, :_N, :_EDGE_DIM], msg[1, :_N, :_EDGE_DIM]],
        axis=1)


# speedup vs baseline: 27.1234x; 2.0076x over previous
import functools

import jax
import jax.numpy as jnp
import numpy as np
from jax import lax
from jax.experimental import pallas as pl
from jax.experimental.pallas import tpu as pltpu
from jax.experimental.pallas import tpu_sc as plsc

_EDGE_DIM = 19
_NUM_GAUSSIANS = _EDGE_DIM - 3
_MASK_NOISE = 0.5
_NUM_GRAPHS = 8

_N = 50000
_E = 1600000

_NR, _NC = 8, 6272
_NPAD2 = _NR * _NC

_NP = 50048
_DUMMY = _NP
_MROWS = 50176
_MC = 24

_SUB = 128
_EPT = 100352
_EP = 16 * _EPT
_NSUB = _EPT // _SUB

_RBF_OFFSET = np.linspace(0.0, 10.0, _NUM_GAUSSIANS)
_RBF_COEFF = -0.5 / float(_RBF_OFFSET[1] - _RBF_OFFSET[0]) ** 2


def _mask_body(in_ref, out_ref):
    x = in_ref[0]
    y = in_ref[1]
    z = in_ref[2]
    b = in_ref[3]
    lig = in_ref[4]
    idxr = in_ref[5]
    shape = x.shape
    masks = [jnp.zeros(shape, jnp.float32), jnp.zeros(shape, jnp.float32)]
    for g in range(_NUM_GRAPHS):
        gm = b == float(g)
        for l in (0, 1):
            chain = gm & (lig == float(l))
            cf = chain.astype(jnp.float32)
            cnt = jnp.sum(cf)
            denom = jnp.maximum(cnt, 1.0)
            cx = jnp.sum(x * cf) / denom
            cy = jnp.sum(y * cf) / denom
            cz = jnp.sum(z * cf) / denom
            dx = x - cx
            dy = y - cy
            dz = z - cz
            dc = jnp.sqrt(dx * dx + dy * dy + dz * dz)
            dc = jnp.where(chain, dc, jnp.inf)
            mn = jnp.min(dc)
            cand = jnp.where(dc == mn, idxr, 3.0e38)
            cidx = jnp.min(cand)
            sel = (idxr == cidx).astype(jnp.float32)
            px = jnp.sum(x * sel)
            py = jnp.sum(y * sel)
            pz = jnp.sum(z * sel)
            ex = x - px
            ey = y - py
            ez = z - pz
            d = jnp.sqrt(ex * ex + ey * ey + ez * ez)
            local = chain & (d < 10.0) & (cnt > 0.0)
            masks[l] = jnp.maximum(masks[l], local.astype(jnp.float32))
    out_ref[0] = masks[0]
    out_ref[1] = masks[1]


_mask_call = pl.pallas_call(
    _mask_body,
    out_shape=jax.ShapeDtypeStruct((2, _NR, _NC), jnp.float32),
)


_CH = 512
_CSUB = _CH // _SUB
_NCH = _EPT // _CH


def _edge_body(rows_hbm, cols_hbm, et0_hbm, et1_hbm, et2_hbm, posv_hbm,
               out_hbm, rows_v, cols_v, cols2_v, e0_v, e1_v, e2_v,
               pr_v, pc_v, feat_v, msg_sh,
               sem_lin, sem_gat, sem_sca):
    c = lax.axis_index("c")
    s = lax.axis_index("s")

    lanes = lax.iota(jnp.int32, 16)
    zeros16 = jnp.zeros((16,), jnp.float32)
    for i8 in range(_SUB // 16):
        for k in range(_MC):
            plsc.store_scatter(
                feat_v, [lanes + i8 * 16, jnp.full((16,), k, jnp.int32)],
                zeros16)
    zrows = _MROWS // 16
    zbase = s * zrows
    zsrc = feat_v.at[pl.ds(0, _SUB)]
    for j in range(zrows // _SUB):
        pltpu.sync_copy(zsrc, msg_sh.at[pl.ds(zbase + j * _SUB, _SUB)])
    pltpu.sync_copy(feat_v.at[pl.ds(0, 64)],
                    msg_sh.at[pl.ds(zbase + (zrows // _SUB) * _SUB, 64)])
    plsc.subcore_barrier()

    zi = jnp.zeros((16,), jnp.int32)
    base = c * 3
    kx = zi + base
    ky = zi + base + 1
    kz = zi + base + 2
    kl = zi + 6

    tile_base = s * _EPT
    tile_row0 = tile_base // _SUB

    def fire_linear(g, slot):
        r0 = tile_row0 + g * _CSUB
        e0 = tile_base + g * _CH
        pltpu.async_copy(rows_hbm.at[pl.ds(r0, _CSUB)],
                         rows_v.at[pl.ds(slot * _CSUB, _CSUB)], sem_lin)
        pltpu.async_copy(cols_hbm.at[pl.ds(r0, _CSUB)],
                         cols_v.at[pl.ds(slot * _CSUB, _CSUB)], sem_lin)
        pltpu.async_copy(et0_hbm.at[pl.ds(e0, _CH)],
                         e0_v.at[pl.ds(slot * _CH, _CH)], sem_lin)
        pltpu.async_copy(et1_hbm.at[pl.ds(e0, _CH)],
                         e1_v.at[pl.ds(slot * _CH, _CH)], sem_lin)
        pltpu.async_copy(et2_hbm.at[pl.ds(e0, _CH)],
                         e2_v.at[pl.ds(slot * _CH, _CH)], sem_lin)

    def wait_linear(slot):
        pltpu.make_async_copy(rows_hbm.at[pl.ds(0, _CSUB)],
                              rows_v.at[pl.ds(slot * _CSUB, _CSUB)],
                              sem_lin).wait()
        pltpu.make_async_copy(cols_hbm.at[pl.ds(0, _CSUB)],
                              cols_v.at[pl.ds(slot * _CSUB, _CSUB)],
                              sem_lin).wait()
        pltpu.make_async_copy(et0_hbm.at[pl.ds(0, _CH)],
                              e0_v.at[pl.ds(slot * _CH, _CH)], sem_lin).wait()
        pltpu.make_async_copy(et1_hbm.at[pl.ds(0, _CH)],
                              e1_v.at[pl.ds(slot * _CH, _CH)], sem_lin).wait()
        pltpu.make_async_copy(et2_hbm.at[pl.ds(0, _CH)],
                              e2_v.at[pl.ds(slot * _CH, _CH)], sem_lin).wait()

    def fire_gathers(lslot, b):
        for j in range(_CSUB):
            d0 = b * _CH + j * _SUB
            pltpu.async_copy(posv_hbm.at[rows_v.at[lslot * _CSUB + j]],
                             pr_v.at[pl.ds(d0, _SUB)], sem_gat)
            pltpu.async_copy(posv_hbm.at[cols_v.at[lslot * _CSUB + j]],
                             pc_v.at[pl.ds(d0, _SUB)], sem_gat)

    def wait_gathers(b):
        for j in range(_CSUB):
            d0 = b * _CH + j * _SUB
            pltpu.make_async_copy(posv_hbm.at[rows_v.at[0]],
                                  pr_v.at[pl.ds(d0, _SUB)], sem_gat).wait()
            pltpu.make_async_copy(posv_hbm.at[cols_v.at[0]],
                                  pc_v.at[pl.ds(d0, _SUB)], sem_gat).wait()

    def fire_scatter(b):
        for j in range(_CSUB):
            d0 = b * _CH + j * _SUB
            pltpu.sync_copy(feat_v.at[pl.ds(d0, _SUB)],
                            msg_sh.at[cols2_v.at[b * _CSUB + j]], add=True)

    def wait_scatter(b):
        del b

    fire_linear(0, 0)
    wait_linear(0)
    fire_gathers(0, 0)
    fire_linear(1, 1)

    def chunk_body(g, carry):
        b = lax.rem(g, 2)
        lg = lax.rem(g, 3)
        wait_gathers(b)

        @pl.when(g + 2 < _NCH)
        def _():
            fire_linear(g + 2, lax.rem(g + 2, 3))

        @pl.when(g + 1 < _NCH)
        def _():
            wait_linear(lax.rem(g + 1, 3))
            fire_gathers(lax.rem(g + 1, 3), 1 - b)

        for i8 in range(_CH // 16):
            j = i8 // 8
            o = (i8 % 8) * 16
            eo = b * _CH + j * _SUB + o
            idx = lanes + eo
            xr = plsc.load_gather(pr_v, [idx, kx])
            yr = plsc.load_gather(pr_v, [idx, ky])
            zr = plsc.load_gather(pr_v, [idx, kz])
            lr = plsc.load_gather(pr_v, [idx, kl])
            xc = plsc.load_gather(pc_v, [idx, kx])
            yc = plsc.load_gather(pc_v, [idx, ky])
            zc = plsc.load_gather(pc_v, [idx, kz])
            lc = plsc.load_gather(pc_v, [idx, kl])
            dx = xr - xc
            dy = yr - yc
            dz = zr - zc
            d2 = dx * dx + dy * dy + dz * dz + 1e-12
            u = lax.bitcast_convert_type(d2, jnp.int32)
            u = jnp.int32(0x5F3759DF) - (u >> 1)
            yv = lax.bitcast_convert_type(u, jnp.float32)
            for _ in range(3):
                yv = yv * (1.5 - 0.5 * d2 * yv * yv)
            d = d2 * yv
            cv = cols_v[lg * _CSUB + j, pl.ds(o, 16)]
            same = lr == lc
            cols2_v[b * _CSUB + j, pl.ds(o, 16)] = jnp.where(
                same, cv, jnp.int32(_DUMMY))
            so = lg * _CH + j * _SUB + o
            e0 = e0_v[pl.ds(so, 16)]
            e1 = e1_v[pl.ds(so, 16)]
            e2 = e2_v[pl.ds(so, 16)]
            plsc.store_scatter(feat_v, [idx, zi], e0)
            plsc.store_scatter(feat_v, [idx, zi + 1], e1)
            plsc.store_scatter(feat_v, [idx, zi + 2], e2)
            for k in range(_NUM_GAUSSIANS):
                t = d - jnp.float32(_RBF_OFFSET[k])
                val = jnp.exp(jnp.float32(_RBF_COEFF) * t * t)
                plsc.store_scatter(feat_v, [idx, zi + (3 + k)], val)

        @pl.when(g > 0)
        def _():
            wait_scatter(1 - b)

        fire_scatter(b)
        return carry

    lax.fori_loop(0, _NCH, chunk_body, 0)
    wait_scatter(lax.rem(_NCH - 1, 2))
    plsc.subcore_barrier()

    orows = _NP // 16
    pltpu.sync_copy(msg_sh.at[pl.ds(s * orows, orows)],
                    out_hbm.at[c, pl.ds(s * orows, orows)])


_edge_call = pl.kernel(
    _edge_body,
    out_type=jax.ShapeDtypeStruct((2, _NP, _MC), jnp.float32),
    mesh=plsc.VectorSubcoreMesh(core_axis_name="c", subcore_axis_name="s"),
    compiler_params=pltpu.CompilerParams(
        needs_layout_passes=False, use_tc_tiling_on_sc=False),
    scratch_types=[
        pltpu.VMEM((3 * _CSUB, _SUB), jnp.int32),
        pltpu.VMEM((3 * _CSUB, _SUB), jnp.int32),
        pltpu.VMEM((2 * _CSUB, _SUB), jnp.int32),
        pltpu.VMEM((3 * _CH,), jnp.float32),
        pltpu.VMEM((3 * _CH,), jnp.float32),
        pltpu.VMEM((3 * _CH,), jnp.float32),
        pltpu.VMEM((2 * _CH, 8), jnp.float32),
        pltpu.VMEM((2 * _CH, 8), jnp.float32),
        pltpu.VMEM((2 * _CH, _MC), jnp.float32),
        pltpu.VMEM_SHARED((_MROWS, _MC), jnp.float32),
        pltpu.SemaphoreType.DMA,
        pltpu.SemaphoreType.DMA,
        pltpu.SemaphoreType.DMA,
    ],
)


def _plane(a, pad_val):
    a = a.astype(jnp.float32)
    return jnp.pad(a, (0, _NPAD2 - _N), constant_values=pad_val).reshape(_NR, _NC)


@jax.jit
def kernel(z, vector_features, edge_index, edge_attr, pos, residue_index,
           is_ligand, mask_interface, batch_idx):
    del z, vector_features, residue_index, mask_interface
    nk = jax.random.key(1)
    n1 = jax.random.normal(jax.random.fold_in(nk, 1), (_N, 3), jnp.float32) * _MASK_NOISE
    n2 = jax.random.normal(jax.random.fold_in(nk, 2), (_N, 3), jnp.float32) * _MASK_NOISE

    planes = jnp.stack([
        _plane(pos[:, 0], 0.0),
        _plane(pos[:, 1], 0.0),
        _plane(pos[:, 2], 0.0),
        _plane(batch_idx, -1.0),
        _plane(is_ligand, -1.0),
        _plane(jnp.arange(_N, dtype=jnp.float32), 3.0e38),
    ])
    masks = _mask_call(planes)
    m1 = (masks[0].reshape(-1)[:_N] > 0.0)[:, None]
    m2 = (masks[1].reshape(-1)[:_N] > 0.0)[:, None]
    pos_v1 = jnp.where(m1, n1, pos)
    pos_v2 = jnp.where(m2, n2, pos)

    ligf = is_ligand.astype(jnp.float32)[:, None]
    pad = jnp.zeros((_N, 1), jnp.float32)
    pv = jnp.concatenate([pos_v1, pos_v2, ligf, pad], axis=1)
    posv = jnp.pad(pv, ((0, _NP - _N), (0, 0)))

    row = edge_index[0].astype(jnp.int32)
    col = edge_index[1].astype(jnp.int32)
    rows1d = jnp.pad(row, (0, _EP - _E)).reshape(_EP // _SUB, _SUB)
    cols1d = jnp.pad(col, (0, _EP - _E),
                     constant_values=_DUMMY).reshape(_EP // _SUB, _SUB)
    et0 = jnp.pad(edge_attr[:, 0], (0, _EP - _E))
    et1 = jnp.pad(edge_attr[:, 1], (0, _EP - _E))
    et2 = jnp.pad(edge_attr[:, 2], (0, _EP - _E))
    msg = _edge_call(rows1d, cols1d, et0, et1, et2, posv)
    return jnp.concatenate(
        [pos_v1, pos_v2, msg[0, :_N, :_EDGE_DIM], msg[1, :_N, :_EDGE_DIM]],
        axis=1)


# speedup vs baseline: 32.5098x; 1.1986x over previous
import functools

import jax
import jax.numpy as jnp
import numpy as np
from jax import lax
from jax.experimental import pallas as pl
from jax.experimental.pallas import tpu as pltpu
from jax.experimental.pallas import tpu_sc as plsc

_EDGE_DIM = 19
_NUM_GAUSSIANS = _EDGE_DIM - 3
_MASK_NOISE = 0.5
_NUM_GRAPHS = 8

_N = 50000
_E = 1600000

_NR, _NC = 8, 6272
_NPAD2 = _NR * _NC

_NP = 50048
_DUMMY = _NP
_MROWS = 50176
_MC = 24

_SUB = 128
_EPT = 100352
_EP = 16 * _EPT
_NSUB = _EPT // _SUB

_RBF_OFFSET = np.linspace(0.0, 10.0, _NUM_GAUSSIANS)
_RBF_COEFF = -0.5 / float(_RBF_OFFSET[1] - _RBF_OFFSET[0]) ** 2


def _mask_body(in_ref, out_ref):
    x = in_ref[0]
    y = in_ref[1]
    z = in_ref[2]
    b = in_ref[3]
    lig = in_ref[4]
    idxr = in_ref[5]
    shape = x.shape
    masks = [jnp.zeros(shape, jnp.float32), jnp.zeros(shape, jnp.float32)]
    for g in range(_NUM_GRAPHS):
        gm = b == float(g)
        for l in (0, 1):
            chain = gm & (lig == float(l))
            cf = chain.astype(jnp.float32)
            cnt = jnp.sum(cf)
            denom = jnp.maximum(cnt, 1.0)
            cx = jnp.sum(x * cf) / denom
            cy = jnp.sum(y * cf) / denom
            cz = jnp.sum(z * cf) / denom
            dx = x - cx
            dy = y - cy
            dz = z - cz
            dc = jnp.sqrt(dx * dx + dy * dy + dz * dz)
            dc = jnp.where(chain, dc, jnp.inf)
            mn = jnp.min(dc)
            cand = jnp.where(dc == mn, idxr, 3.0e38)
            cidx = jnp.min(cand)
            sel = (idxr == cidx).astype(jnp.float32)
            px = jnp.sum(x * sel)
            py = jnp.sum(y * sel)
            pz = jnp.sum(z * sel)
            ex = x - px
            ey = y - py
            ez = z - pz
            d = jnp.sqrt(ex * ex + ey * ey + ez * ez)
            local = chain & (d < 10.0) & (cnt > 0.0)
            masks[l] = jnp.maximum(masks[l], local.astype(jnp.float32))
    out_ref[0] = masks[0]
    out_ref[1] = masks[1]


_mask_call = pl.pallas_call(
    _mask_body,
    out_shape=jax.ShapeDtypeStruct((2, _NR, _NC), jnp.float32),
)


_CH = 512
_CSUB = _CH // _SUB
_NCH = _EPT // _CH


def _edge_body(rows_hbm, cols_hbm, et0_hbm, et1_hbm, et2_hbm, posv_hbm,
               out_hbm, rows_v, cols_v, cols2_v, e0_v, e1_v, e2_v,
               pr_v, pc_v, feat_v, msg_sh,
               sem_lin, sem_gat, sem_sca):
    c = lax.axis_index("c")
    s = lax.axis_index("s")

    lanes = lax.iota(jnp.int32, 16)
    zeros16 = jnp.zeros((16,), jnp.float32)
    for i8 in range(_SUB // 16):
        for k in range(_MC):
            plsc.store_scatter(
                feat_v, [lanes + i8 * 16, jnp.full((16,), k, jnp.int32)],
                zeros16)
    zrows = _MROWS // 16
    zbase = s * zrows
    zsrc = feat_v.at[pl.ds(0, _SUB)]
    for j in range(zrows // _SUB):
        pltpu.sync_copy(zsrc, msg_sh.at[pl.ds(zbase + j * _SUB, _SUB)])
    pltpu.sync_copy(feat_v.at[pl.ds(0, 64)],
                    msg_sh.at[pl.ds(zbase + (zrows // _SUB) * _SUB, 64)])
    plsc.subcore_barrier()

    zi = jnp.zeros((16,), jnp.int32)
    base = c * 3
    kx = zi + base
    ky = zi + base + 1
    kz = zi + base + 2
    kl = zi + 6

    tile_base = s * _EPT
    tile_row0 = tile_base // _SUB

    def fire_linear(g, slot):
        r0 = tile_row0 + g * _CSUB
        e0 = tile_base + g * _CH
        pltpu.async_copy(rows_hbm.at[pl.ds(r0, _CSUB)],
                         rows_v.at[pl.ds(slot * _CSUB, _CSUB)], sem_lin)
        pltpu.async_copy(cols_hbm.at[pl.ds(r0, _CSUB)],
                         cols_v.at[pl.ds(slot * _CSUB, _CSUB)], sem_lin)
        pltpu.async_copy(et0_hbm.at[pl.ds(e0, _CH)],
                         e0_v.at[pl.ds(slot * _CH, _CH)], sem_lin)
        pltpu.async_copy(et1_hbm.at[pl.ds(e0, _CH)],
                         e1_v.at[pl.ds(slot * _CH, _CH)], sem_lin)
        pltpu.async_copy(et2_hbm.at[pl.ds(e0, _CH)],
                         e2_v.at[pl.ds(slot * _CH, _CH)], sem_lin)

    def wait_linear(slot):
        pltpu.make_async_copy(rows_hbm.at[pl.ds(0, _CSUB)],
                              rows_v.at[pl.ds(slot * _CSUB, _CSUB)],
                              sem_lin).wait()
        pltpu.make_async_copy(cols_hbm.at[pl.ds(0, _CSUB)],
                              cols_v.at[pl.ds(slot * _CSUB, _CSUB)],
                              sem_lin).wait()
        pltpu.make_async_copy(et0_hbm.at[pl.ds(0, _CH)],
                              e0_v.at[pl.ds(slot * _CH, _CH)], sem_lin).wait()
        pltpu.make_async_copy(et1_hbm.at[pl.ds(0, _CH)],
                              e1_v.at[pl.ds(slot * _CH, _CH)], sem_lin).wait()
        pltpu.make_async_copy(et2_hbm.at[pl.ds(0, _CH)],
                              e2_v.at[pl.ds(slot * _CH, _CH)], sem_lin).wait()

    def fire_gathers(lslot, b):
        for j in range(_CSUB):
            d0 = b * _CH + j * _SUB
            pltpu.async_copy(posv_hbm.at[rows_v.at[lslot * _CSUB + j]],
                             pr_v.at[pl.ds(d0, _SUB)], sem_gat)
            pltpu.async_copy(posv_hbm.at[cols_v.at[lslot * _CSUB + j]],
                             pc_v.at[pl.ds(d0, _SUB)], sem_gat)

    def wait_gathers(b):
        for j in range(_CSUB):
            d0 = b * _CH + j * _SUB
            pltpu.make_async_copy(posv_hbm.at[rows_v.at[0]],
                                  pr_v.at[pl.ds(d0, _SUB)], sem_gat).wait()
            pltpu.make_async_copy(posv_hbm.at[cols_v.at[0]],
                                  pc_v.at[pl.ds(d0, _SUB)], sem_gat).wait()

    def fire_scatter(b):
        for j in range(_CSUB):
            d0 = b * _CH + j * _SUB
            pltpu.async_copy(feat_v.at[pl.ds(d0, _SUB)],
                             msg_sh.at[cols2_v.at[b * _CSUB + j]], sem_sca,
                             add=True)

    def wait_scatter(b):
        for j in range(_CSUB):
            d0 = b * _CH + j * _SUB
            pltpu.make_async_copy(feat_v.at[pl.ds(d0, _SUB)],
                                  msg_sh.at[cols2_v.at[0]], sem_sca).wait()

    fire_linear(0, 0)
    wait_linear(0)
    fire_gathers(0, 0)
    fire_linear(1, 1)

    def chunk_body(g, carry):
        b = lax.rem(g, 2)
        lg = lax.rem(g, 3)
        wait_gathers(b)

        @pl.when(g + 2 < _NCH)
        def _():
            fire_linear(g + 2, lax.rem(g + 2, 3))

        @pl.when(g + 1 < _NCH)
        def _():
            wait_linear(lax.rem(g + 1, 3))
            fire_gathers(lax.rem(g + 1, 3), 1 - b)

        for i8 in range(_CH // 16):
            j = i8 // 8
            o = (i8 % 8) * 16
            eo = b * _CH + j * _SUB + o
            idx = lanes + eo
            xr = plsc.load_gather(pr_v, [idx, kx])
            yr = plsc.load_gather(pr_v, [idx, ky])
            zr = plsc.load_gather(pr_v, [idx, kz])
            lr = plsc.load_gather(pr_v, [idx, kl])
            xc = plsc.load_gather(pc_v, [idx, kx])
            yc = plsc.load_gather(pc_v, [idx, ky])
            zc = plsc.load_gather(pc_v, [idx, kz])
            lc = plsc.load_gather(pc_v, [idx, kl])
            dx = xr - xc
            dy = yr - yc
            dz = zr - zc
            d2 = dx * dx + dy * dy + dz * dz + 1e-12
            u = lax.bitcast_convert_type(d2, jnp.int32)
            u = jnp.int32(0x5F3759DF) - (u >> 1)
            yv = lax.bitcast_convert_type(u, jnp.float32)
            for _ in range(3):
                yv = yv * (1.5 - 0.5 * d2 * yv * yv)
            d = d2 * yv
            cv = cols_v[lg * _CSUB + j, pl.ds(o, 16)]
            same = lr == lc
            cols2_v[b * _CSUB + j, pl.ds(o, 16)] = jnp.where(
                same, cv, jnp.int32(_DUMMY))
            so = lg * _CH + j * _SUB + o
            e0 = e0_v[pl.ds(so, 16)]
            e1 = e1_v[pl.ds(so, 16)]
            e2 = e2_v[pl.ds(so, 16)]
            plsc.store_scatter(feat_v, [idx, zi], e0)
            plsc.store_scatter(feat_v, [idx, zi + 1], e1)
            plsc.store_scatter(feat_v, [idx, zi + 2], e2)
            for k in range(_NUM_GAUSSIANS):
                t = d - jnp.float32(_RBF_OFFSET[k])
                val = jnp.exp(jnp.float32(_RBF_COEFF) * t * t)
                plsc.store_scatter(feat_v, [idx, zi + (3 + k)], val)

        @pl.when(g > 0)
        def _():
            wait_scatter(1 - b)

        fire_scatter(b)
        return carry

    lax.fori_loop(0, _NCH, chunk_body, 0)
    wait_scatter(lax.rem(_NCH - 1, 2))
    plsc.subcore_barrier()

    orows = _NP // 16
    pltpu.sync_copy(msg_sh.at[pl.ds(s * orows, orows)],
                    out_hbm.at[c, pl.ds(s * orows, orows)])


_edge_call = pl.kernel(
    _edge_body,
    out_type=jax.ShapeDtypeStruct((2, _NP, _MC), jnp.float32),
    mesh=plsc.VectorSubcoreMesh(core_axis_name="c", subcore_axis_name="s"),
    compiler_params=pltpu.CompilerParams(
        needs_layout_passes=False, use_tc_tiling_on_sc=False),
    scratch_types=[
        pltpu.VMEM((3 * _CSUB, _SUB), jnp.int32),
        pltpu.VMEM((3 * _CSUB, _SUB), jnp.int32),
        pltpu.VMEM((2 * _CSUB, _SUB), jnp.int32),
        pltpu.VMEM((3 * _CH,), jnp.float32),
        pltpu.VMEM((3 * _CH,), jnp.float32),
        pltpu.VMEM((3 * _CH,), jnp.float32),
        pltpu.VMEM((2 * _CH, 8), jnp.float32),
        pltpu.VMEM((2 * _CH, 8), jnp.float32),
        pltpu.VMEM((2 * _CH, _MC), jnp.float32),
        pltpu.VMEM_SHARED((_MROWS, _MC), jnp.float32),
        pltpu.SemaphoreType.DMA,
        pltpu.SemaphoreType.DMA,
        pltpu.SemaphoreType.DMA,
    ],
)


def _plane(a, pad_val):
    a = a.astype(jnp.float32)
    return jnp.pad(a, (0, _NPAD2 - _N), constant_values=pad_val).reshape(_NR, _NC)


@jax.jit
def kernel(z, vector_features, edge_index, edge_attr, pos, residue_index,
           is_ligand, mask_interface, batch_idx):
    del z, vector_features, residue_index, mask_interface
    nk = jax.random.key(1)
    n1 = jax.random.normal(jax.random.fold_in(nk, 1), (_N, 3), jnp.float32) * _MASK_NOISE
    n2 = jax.random.normal(jax.random.fold_in(nk, 2), (_N, 3), jnp.float32) * _MASK_NOISE

    planes = jnp.stack([
        _plane(pos[:, 0], 0.0),
        _plane(pos[:, 1], 0.0),
        _plane(pos[:, 2], 0.0),
        _plane(batch_idx, -1.0),
        _plane(is_ligand, -1.0),
        _plane(jnp.arange(_N, dtype=jnp.float32), 3.0e38),
    ])
    masks = _mask_call(planes)
    m1 = (masks[0].reshape(-1)[:_N] > 0.0)[:, None]
    m2 = (masks[1].reshape(-1)[:_N] > 0.0)[:, None]
    pos_v1 = jnp.where(m1, n1, pos)
    pos_v2 = jnp.where(m2, n2, pos)

    ligf = is_ligand.astype(jnp.float32)[:, None]
    pad = jnp.zeros((_N, 1), jnp.float32)
    pv = jnp.concatenate([pos_v1, pos_v2, ligf, pad], axis=1)
    posv = jnp.pad(pv, ((0, _NP - _N), (0, 0)))

    row = edge_index[0].astype(jnp.int32)
    col = edge_index[1].astype(jnp.int32)
    rows1d = jnp.pad(row, (0, _EP - _E)).reshape(_EP // _SUB, _SUB)
    cols1d = jnp.pad(col, (0, _EP - _E),
                     constant_values=_DUMMY).reshape(_EP // _SUB, _SUB)
    et0 = jnp.pad(edge_attr[:, 0], (0, _EP - _E))
    et1 = jnp.pad(edge_attr[:, 1], (0, _EP - _E))
    et2 = jnp.pad(edge_attr[:, 2], (0, _EP - _E))
    msg = _edge_call(rows1d, cols1d, et0, et1, et2, posv)
    return jnp.concatenate(
        [pos_v1, pos_v2, msg[0, :_N, :_EDGE_DIM], msg[1, :_N, :_EDGE_DIM]],
        axis=1)


# speedup vs baseline: 32.6215x; 1.0034x over previous
import functools

import jax
import jax.numpy as jnp
import numpy as np
from jax import lax
from jax.experimental import pallas as pl
from jax.experimental.pallas import tpu as pltpu
from jax.experimental.pallas import tpu_sc as plsc

_EDGE_DIM = 19
_NUM_GAUSSIANS = _EDGE_DIM - 3
_MASK_NOISE = 0.5
_NUM_GRAPHS = 8

_N = 50000
_E = 1600000

_NR, _NC = 8, 6272
_NPAD2 = _NR * _NC

_NP = 50048
_DUMMY = _NP
_MROWS = 50176
_MC = 24

_SUB = 128
_EPT = 100352
_EP = 16 * _EPT
_NSUB = _EPT // _SUB

_RBF_OFFSET = np.linspace(0.0, 10.0, _NUM_GAUSSIANS)
_RBF_COEFF = -0.5 / float(_RBF_OFFSET[1] - _RBF_OFFSET[0]) ** 2


def _mask_body(in_ref, out_ref):
    x = in_ref[0]
    y = in_ref[1]
    z = in_ref[2]
    b = in_ref[3]
    lig = in_ref[4]
    idxr = in_ref[5]
    shape = x.shape
    masks = [jnp.zeros(shape, jnp.float32), jnp.zeros(shape, jnp.float32)]
    for g in range(_NUM_GRAPHS):
        gm = b == float(g)
        for l in (0, 1):
            chain = gm & (lig == float(l))
            cf = chain.astype(jnp.float32)
            cnt = jnp.sum(cf)
            denom = jnp.maximum(cnt, 1.0)
            cx = jnp.sum(x * cf) / denom
            cy = jnp.sum(y * cf) / denom
            cz = jnp.sum(z * cf) / denom
            dx = x - cx
            dy = y - cy
            dz = z - cz
            dc = jnp.sqrt(dx * dx + dy * dy + dz * dz)
            dc = jnp.where(chain, dc, jnp.inf)
            mn = jnp.min(dc)
            cand = jnp.where(dc == mn, idxr, 3.0e38)
            cidx = jnp.min(cand)
            sel = (idxr == cidx).astype(jnp.float32)
            px = jnp.sum(x * sel)
            py = jnp.sum(y * sel)
            pz = jnp.sum(z * sel)
            ex = x - px
            ey = y - py
            ez = z - pz
            d = jnp.sqrt(ex * ex + ey * ey + ez * ez)
            local = chain & (d < 10.0) & (cnt > 0.0)
            masks[l] = jnp.maximum(masks[l], local.astype(jnp.float32))
    out_ref[0] = masks[0]
    out_ref[1] = masks[1]


_mask_call = pl.pallas_call(
    _mask_body,
    out_shape=jax.ShapeDtypeStruct((2, _NR, _NC), jnp.float32),
)


_CH = 512
_CSUB = _CH // _SUB
_NCH = _EPT // _CH


def _edge_body(rows_hbm, cols_hbm, et0_hbm, et1_hbm, et2_hbm, posv_hbm,
               out_hbm, rows_v, cols_v, cols2_v, e0_v, e1_v, e2_v,
               pr_v, pc_v, feat_v, msg_sh,
               sem_lin, sem_gat, sem_sca):
    c = lax.axis_index("c")
    s = lax.axis_index("s")

    lanes = lax.iota(jnp.int32, 16)
    zeros16 = jnp.zeros((16,), jnp.float32)
    for i8 in range(_SUB // 16):
        for k in range(_MC):
            plsc.store_scatter(
                feat_v, [lanes + i8 * 16, jnp.full((16,), k, jnp.int32)],
                zeros16)
    zrows = _MROWS // 16
    zbase = s * zrows
    zsrc = feat_v.at[pl.ds(0, _SUB)]
    for j in range(zrows // _SUB):
        pltpu.sync_copy(zsrc, msg_sh.at[pl.ds(zbase + j * _SUB, _SUB)])
    pltpu.sync_copy(feat_v.at[pl.ds(0, 64)],
                    msg_sh.at[pl.ds(zbase + (zrows // _SUB) * _SUB, 64)])
    plsc.subcore_barrier()

    zi = jnp.zeros((16,), jnp.int32)
    base = c * 3
    kx = zi + base
    ky = zi + base + 1
    kz = zi + base + 2
    kl = zi + 6

    tile_base = s * _EPT
    tile_row0 = tile_base // _SUB

    def fire_linear(g, slot):
        r0 = tile_row0 + g * _CSUB
        e0 = tile_base + g * _CH
        pltpu.async_copy(rows_hbm.at[pl.ds(r0, _CSUB)],
                         rows_v.at[pl.ds(slot * _CSUB, _CSUB)], sem_lin)
        pltpu.async_copy(cols_hbm.at[pl.ds(r0, _CSUB)],
                         cols_v.at[pl.ds(slot * _CSUB, _CSUB)], sem_lin)
        pltpu.async_copy(et0_hbm.at[pl.ds(e0, _CH)],
                         e0_v.at[pl.ds(slot * _CH, _CH)], sem_lin)
        pltpu.async_copy(et1_hbm.at[pl.ds(e0, _CH)],
                         e1_v.at[pl.ds(slot * _CH, _CH)], sem_lin)
        pltpu.async_copy(et2_hbm.at[pl.ds(e0, _CH)],
                         e2_v.at[pl.ds(slot * _CH, _CH)], sem_lin)

    def wait_linear(slot):
        pltpu.make_async_copy(rows_hbm.at[pl.ds(0, _CSUB)],
                              rows_v.at[pl.ds(slot * _CSUB, _CSUB)],
                              sem_lin).wait()
        pltpu.make_async_copy(cols_hbm.at[pl.ds(0, _CSUB)],
                              cols_v.at[pl.ds(slot * _CSUB, _CSUB)],
                              sem_lin).wait()
        pltpu.make_async_copy(et0_hbm.at[pl.ds(0, _CH)],
                              e0_v.at[pl.ds(slot * _CH, _CH)], sem_lin).wait()
        pltpu.make_async_copy(et1_hbm.at[pl.ds(0, _CH)],
                              e1_v.at[pl.ds(slot * _CH, _CH)], sem_lin).wait()
        pltpu.make_async_copy(et2_hbm.at[pl.ds(0, _CH)],
                              e2_v.at[pl.ds(slot * _CH, _CH)], sem_lin).wait()

    def fire_gathers(lslot, b):
        for j in range(_CSUB):
            d0 = b * _CH + j * _SUB
            pltpu.async_copy(posv_hbm.at[rows_v.at[lslot * _CSUB + j]],
                             pr_v.at[pl.ds(d0, _SUB)], sem_gat)
            pltpu.async_copy(posv_hbm.at[cols_v.at[lslot * _CSUB + j]],
                             pc_v.at[pl.ds(d0, _SUB)], sem_gat)

    def wait_gathers(b):
        for j in range(_CSUB):
            d0 = b * _CH + j * _SUB
            pltpu.make_async_copy(posv_hbm.at[rows_v.at[0]],
                                  pr_v.at[pl.ds(d0, _SUB)], sem_gat).wait()
            pltpu.make_async_copy(posv_hbm.at[cols_v.at[0]],
                                  pc_v.at[pl.ds(d0, _SUB)], sem_gat).wait()

    def fire_scatter(b):
        for j in range(_CSUB):
            d0 = b * _CH + j * _SUB
            pltpu.async_copy(feat_v.at[pl.ds(d0, _SUB)],
                             msg_sh.at[cols2_v.at[b * _CSUB + j]], sem_sca,
                             add=True)

    def wait_scatter(b):
        for j in range(_CSUB):
            d0 = b * _CH + j * _SUB
            pltpu.make_async_copy(feat_v.at[pl.ds(d0, _SUB)],
                                  msg_sh.at[cols2_v.at[0]], sem_sca).wait()

    fire_linear(0, 0)
    wait_linear(0)
    fire_gathers(0, 0)
    fire_linear(1, 1)

    def chunk_body(g, carry):
        b = lax.rem(g, 2)
        lg = lax.rem(g, 3)
        wait_gathers(b)

        @pl.when(g + 2 < _NCH)
        def _():
            fire_linear(g + 2, lax.rem(g + 2, 3))

        @pl.when(g + 1 < _NCH)
        def _():
            wait_linear(lax.rem(g + 1, 3))
            fire_gathers(lax.rem(g + 1, 3), 1 - b)

        for i8 in range(_CH // 16):
            j = i8 // 8
            o = (i8 % 8) * 16
            eo = b * _CH + j * _SUB + o
            idx = lanes + eo
            xr = plsc.load_gather(pr_v, [idx, kx])
            yr = plsc.load_gather(pr_v, [idx, ky])
            zr = plsc.load_gather(pr_v, [idx, kz])
            lr = plsc.load_gather(pr_v, [idx, kl])
            xc = plsc.load_gather(pc_v, [idx, kx])
            yc = plsc.load_gather(pc_v, [idx, ky])
            zc = plsc.load_gather(pc_v, [idx, kz])
            lc = plsc.load_gather(pc_v, [idx, kl])
            dx = xr - xc
            dy = yr - yc
            dz = zr - zc
            d2 = dx * dx + dy * dy + dz * dz + 1e-12
            u = lax.bitcast_convert_type(d2, jnp.int32)
            u = jnp.int32(0x5F3759DF) - (u >> 1)
            yv = lax.bitcast_convert_type(u, jnp.float32)
            for _ in range(2):
                yv = yv * (1.5 - 0.5 * d2 * yv * yv)
            d = d2 * yv
            cv = cols_v[lg * _CSUB + j, pl.ds(o, 16)]
            same = lr == lc
            cols2_v[b * _CSUB + j, pl.ds(o, 16)] = jnp.where(
                same, cv, jnp.int32(_DUMMY))
            so = lg * _CH + j * _SUB + o
            e0 = e0_v[pl.ds(so, 16)]
            e1 = e1_v[pl.ds(so, 16)]
            e2 = e2_v[pl.ds(so, 16)]
            plsc.store_scatter(feat_v, [idx, zi], e0)
            plsc.store_scatter(feat_v, [idx, zi + 1], e1)
            plsc.store_scatter(feat_v, [idx, zi + 2], e2)
            for k in range(_NUM_GAUSSIANS):
                t = d - jnp.float32(_RBF_OFFSET[k])
                val = jnp.exp(jnp.float32(_RBF_COEFF) * t * t)
                plsc.store_scatter(feat_v, [idx, zi + (3 + k)], val)

        @pl.when(g > 0)
        def _():
            wait_scatter(1 - b)

        fire_scatter(b)
        return carry

    lax.fori_loop(0, _NCH, chunk_body, 0)
    wait_scatter(lax.rem(_NCH - 1, 2))
    plsc.subcore_barrier()

    orows = _NP // 16
    pltpu.sync_copy(msg_sh.at[pl.ds(s * orows, orows)],
                    out_hbm.at[c, pl.ds(s * orows, orows)])


_edge_call = pl.kernel(
    _edge_body,
    out_type=jax.ShapeDtypeStruct((2, _NP, _MC), jnp.float32),
    mesh=plsc.VectorSubcoreMesh(core_axis_name="c", subcore_axis_name="s"),
    compiler_params=pltpu.CompilerParams(
        needs_layout_passes=False, use_tc_tiling_on_sc=False),
    scratch_types=[
        pltpu.VMEM((3 * _CSUB, _SUB), jnp.int32),
        pltpu.VMEM((3 * _CSUB, _SUB), jnp.int32),
        pltpu.VMEM((2 * _CSUB, _SUB), jnp.int32),
        pltpu.VMEM((3 * _CH,), jnp.float32),
        pltpu.VMEM((3 * _CH,), jnp.float32),
        pltpu.VMEM((3 * _CH,), jnp.float32),
        pltpu.VMEM((2 * _CH, 8), jnp.float32),
        pltpu.VMEM((2 * _CH, 8), jnp.float32),
        pltpu.VMEM((2 * _CH, _MC), jnp.float32),
        pltpu.VMEM_SHARED((_MROWS, _MC), jnp.float32),
        pltpu.SemaphoreType.DMA,
        pltpu.SemaphoreType.DMA,
        pltpu.SemaphoreType.DMA,
    ],
)


def _plane(a, pad_val):
    a = a.astype(jnp.float32)
    return jnp.pad(a, (0, _NPAD2 - _N), constant_values=pad_val).reshape(_NR, _NC)


@jax.jit
def kernel(z, vector_features, edge_index, edge_attr, pos, residue_index,
           is_ligand, mask_interface, batch_idx):
    del z, vector_features, residue_index, mask_interface
    nk = jax.random.key(1)
    n1 = jax.random.normal(jax.random.fold_in(nk, 1), (_N, 3), jnp.float32) * _MASK_NOISE
    n2 = jax.random.normal(jax.random.fold_in(nk, 2), (_N, 3), jnp.float32) * _MASK_NOISE

    planes = jnp.stack([
        _plane(pos[:, 0], 0.0),
        _plane(pos[:, 1], 0.0),
        _plane(pos[:, 2], 0.0),
        _plane(batch_idx, -1.0),
        _plane(is_ligand, -1.0),
        _plane(jnp.arange(_N, dtype=jnp.float32), 3.0e38),
    ])
    masks = _mask_call(planes)
    m1 = (masks[0].reshape(-1)[:_N] > 0.0)[:, None]
    m2 = (masks[1].reshape(-1)[:_N] > 0.0)[:, None]
    pos_v1 = jnp.where(m1, n1, pos)
    pos_v2 = jnp.where(m2, n2, pos)

    ligf = is_ligand.astype(jnp.float32)[:, None]
    pad = jnp.zeros((_N, 1), jnp.float32)
    pv = jnp.concatenate([pos_v1, pos_v2, ligf, pad], axis=1)
    posv = jnp.pad(pv, ((0, _NP - _N), (0, 0)))

    row = edge_index[0].astype(jnp.int32)
    col = edge_index[1].astype(jnp.int32)
    rows1d = jnp.pad(row, (0, _EP - _E)).reshape(_EP // _SUB, _SUB)
    cols1d = jnp.pad(col, (0, _EP - _E),
                     constant_values=_DUMMY).reshape(_EP // _SUB, _SUB)
    et0 = jnp.pad(edge_attr[:, 0], (0, _EP - _E))
    et1 = jnp.pad(edge_attr[:, 1], (0, _EP - _E))
    et2 = jnp.pad(edge_attr[:, 2], (0, _EP - _E))
    msg = _edge_call(rows1d, cols1d, et0, et1, et2, posv)
    return jnp.concatenate(
        [pos_v1, pos_v2, msg[0, :_N, :_EDGE_DIM], msg[1, :_N, :_EDGE_DIM]],
        axis=1)
